# Initial kernel scaffold; baseline (speedup 1.0000x reference)
import functools

import jax
import jax.numpy as jnp
from jax import lax
from jax.experimental import pallas as pl
from jax.experimental.pallas import tpu as pltpu
from jax.experimental.pallas import tpu_sc as plsc

N = 10000
E = 320000
D = 128
W = 144
NPAD = 10240
NC = 2
NS = 16
NW = NC * NS
EP_W = E // NW
CH = 80
NCH = EP_W // CH
RPT = NPAD // NS
MM_BLK = 1024


def _mm_body(x_ref, ws_ref, wt_ref, src_ref, tgt_ref):
    xb = x_ref[...]
    s = jnp.dot(xb, ws_ref[...], preferred_element_type=jnp.float32)
    t = jnp.dot(xb, wt_ref[...], preferred_element_type=jnp.float32)
    lane = lax.broadcasted_iota(jnp.int32, (MM_BLK, W - D), 1)
    one_col = jnp.where(lane == 0, 1.0, 0.0).astype(jnp.float32)
    src_ref[...] = jnp.concatenate([s, one_col], axis=1)
    tgt_ref[...] = jnp.concatenate([t, jnp.zeros_like(one_col)], axis=1)


_mm_call = pl.pallas_call(
    _mm_body,
    grid=(NPAD // MM_BLK,),
    in_specs=[
        pl.BlockSpec((MM_BLK, D), lambda i: (i, 0)),
        pl.BlockSpec((D, D), lambda i: (0, 0)),
        pl.BlockSpec((D, D), lambda i: (0, 0)),
    ],
    out_specs=[
        pl.BlockSpec((MM_BLK, W), lambda i: (i, 0)),
        pl.BlockSpec((MM_BLK, W), lambda i: (i, 0)),
    ],
    out_shape=[
        jax.ShapeDtypeStruct((NPAD, W), jnp.float32),
        jax.ShapeDtypeStruct((NPAD, W), jnp.float32),
    ],
)


_mesh = plsc.VectorSubcoreMesh(core_axis_name="c", subcore_axis_name="s")


@functools.partial(
    pl.kernel,
    mesh=_mesh,
    out_type=jax.ShapeDtypeStruct((NC, NPAD, W), jnp.float32),
    scratch_types=[
        pltpu.VMEM_SHARED((NPAD, W), jnp.float32),
        pltpu.VMEM((CH, W), jnp.float32),
        pltpu.VMEM((CH,), jnp.int32),
        pltpu.VMEM((CH,), jnp.int32),
        pltpu.SemaphoreType.DMA,
    ],
)
def _sc_kernel(srcx, sidx_h, didx_h, out, acc, rows_v, si_v, di_v, sem):
    c = lax.axis_index("c")
    s = lax.axis_index("s")
    wid = c * NS + s

    z16 = jnp.zeros((16,), jnp.float32)

    def _zrow(i, carry):
        for j in range(W // 16):
            rows_v[i, pl.ds(j * 16, 16)] = z16
        return carry

    lax.fori_loop(0, CH, _zrow, 0)
    for k in range(RPT // CH):
        pltpu.sync_copy(rows_v, acc.at[pl.ds(s * RPT + k * CH, CH)])
    plsc.subcore_barrier()

    def _body(i, carry):
        base = wid * EP_W + i * CH
        pltpu.sync_copy(sidx_h.at[pl.ds(base, CH)], si_v)
        pltpu.sync_copy(didx_h.at[pl.ds(base, CH)], di_v)
        pltpu.async_copy(srcx.at[si_v], rows_v, sem).wait()
        pltpu.sync_copy(rows_v, acc.at[di_v], add=True)
        return carry

    lax.fori_loop(0, NCH, _body, 0)
    plsc.subcore_barrier()

    pltpu.sync_copy(acc.at[pl.ds(s * RPT, RPT)], out.at[c, pl.ds(s * RPT, RPT)])


def _combine_body(p_ref, t_ref, o_ref):
    p = p_ref[...]
    total = p[0] + p[1]
    lane = lax.broadcasted_iota(jnp.int32, total.shape, 1)
    cnt = jnp.sum(jnp.where(lane == D, total, 0.0), axis=1, keepdims=True)
    mean = total * (1.0 / jnp.maximum(cnt, 1.0))
    o_ref[...] = jnp.maximum(t_ref[...] + mean, 0.0)


_combine_call = pl.pallas_call(
    _combine_body,
    grid=(NPAD // MM_BLK,),
    in_specs=[
        pl.BlockSpec((NC, MM_BLK, W), lambda i: (0, i, 0)),
        pl.BlockSpec((MM_BLK, W), lambda i: (i, 0)),
    ],
    out_specs=pl.BlockSpec((MM_BLK, W), lambda i: (i, 0)),
    out_shape=jax.ShapeDtypeStruct((NPAD, W), jnp.float32),
)


def kernel(x, edge_index, W_source, W_target):
    xpad = jnp.pad(x, ((0, NPAD - N), (0, 0)))
    srcx, tgtx = _mm_call(xpad, W_source, W_target)
    sidx = edge_index[0].reshape(-1)
    didx = edge_index[1].reshape(-1)
    partials = _sc_kernel(srcx, sidx, didx)
    y = _combine_call(partials, tgtx)
    return y[:N, :D]


# speedup vs baseline: 5.0724x; 5.0724x over previous
import functools

import jax
import jax.numpy as jnp
from jax import lax
from jax.experimental import pallas as pl
from jax.experimental.pallas import tpu as pltpu
from jax.experimental.pallas import tpu_sc as plsc

N = 10000
E = 320000
D = 128
W = 144
NPAD = 10240
NC = 2
NS = 16
NW = NC * NS
EP_W = E // NW
CH = 80
NCH = EP_W // CH
RPT = NPAD // NS
MM_BLK = 1024


def _mm_body(x_ref, ws_ref, wt_ref, src_ref, tgt_ref):
    xb = x_ref[...]
    s = jnp.dot(xb, ws_ref[...], preferred_element_type=jnp.float32)
    t = jnp.dot(xb, wt_ref[...], preferred_element_type=jnp.float32)
    lane = lax.broadcasted_iota(jnp.int32, (MM_BLK, W - D), 1)
    one_col = jnp.where(lane == 0, 1.0, 0.0).astype(jnp.float32)
    src_ref[...] = jnp.concatenate([s, one_col], axis=1)
    tgt_ref[...] = jnp.concatenate([t, jnp.zeros_like(one_col)], axis=1)


_mm_call = pl.pallas_call(
    _mm_body,
    grid=(NPAD // MM_BLK,),
    in_specs=[
        pl.BlockSpec((MM_BLK, D), lambda i: (i, 0)),
        pl.BlockSpec((D, D), lambda i: (0, 0)),
        pl.BlockSpec((D, D), lambda i: (0, 0)),
    ],
    out_specs=[
        pl.BlockSpec((MM_BLK, W), lambda i: (i, 0)),
        pl.BlockSpec((MM_BLK, W), lambda i: (i, 0)),
    ],
    out_shape=[
        jax.ShapeDtypeStruct((NPAD, W), jnp.float32),
        jax.ShapeDtypeStruct((NPAD, W), jnp.float32),
    ],
)


def _sc_body(srcx, sidx_h, didx_h, out, acc, rows_v, si_v, di_v, sem):
    c = lax.axis_index("c")
    s = lax.axis_index("s")
    wid = c * NS + s

    z16 = jnp.zeros((16,), jnp.float32)

    def _zrow(i, carry):
        for j in range(W // 16):
            rows_v[i, pl.ds(j * 16, 16)] = z16
        return carry

    lax.fori_loop(0, CH, _zrow, 0)
    for k in range(RPT // CH):
        pltpu.sync_copy(rows_v, acc.at[pl.ds(s * RPT + k * CH, CH)])
    plsc.subcore_barrier()

    def _body(i, carry):
        base = wid * EP_W + i * CH
        pltpu.sync_copy(sidx_h.at[pl.ds(base, CH)], si_v)
        pltpu.sync_copy(didx_h.at[pl.ds(base, CH)], di_v)
        pltpu.async_copy(srcx.at[si_v], rows_v, sem).wait()
        pltpu.sync_copy(rows_v, acc.at[di_v], add=True)
        return carry

    lax.fori_loop(0, NCH, _body, 0)
    plsc.subcore_barrier()

    pltpu.sync_copy(acc.at[pl.ds(s * RPT, RPT)], out.at[c, pl.ds(s * RPT, RPT)])


def _combine_body(p_ref, t_ref, o_ref):
    p = p_ref[...]
    total = p[0] + p[1]
    lane = lax.broadcasted_iota(jnp.int32, total.shape, 1)
    cnt = jnp.sum(jnp.where(lane == D, total, 0.0), axis=1, keepdims=True)
    mean = total * (1.0 / jnp.maximum(cnt, 1.0))
    o_ref[...] = jnp.maximum(t_ref[...] + mean, 0.0)


_combine_call = pl.pallas_call(
    _combine_body,
    grid=(NPAD // MM_BLK,),
    in_specs=[
        pl.BlockSpec((NC, MM_BLK, W), lambda i: (0, i, 0)),
        pl.BlockSpec((MM_BLK, W), lambda i: (i, 0)),
    ],
    out_specs=pl.BlockSpec((MM_BLK, W), lambda i: (i, 0)),
    out_shape=jax.ShapeDtypeStruct((NPAD, W), jnp.float32),
)


@functools.cache
def _sc_call():
    mesh = plsc.VectorSubcoreMesh(
        core_axis_name="c", subcore_axis_name="s", num_cores=NC, num_subcores=NS
    )
    return pl.kernel(
        _sc_body,
        mesh=mesh,
        out_type=jax.ShapeDtypeStruct((NC, NPAD, W), jnp.float32),
        scratch_types=[
            pltpu.VMEM_SHARED((NPAD, W), jnp.float32),
            pltpu.VMEM((CH, W), jnp.float32),
            pltpu.VMEM((CH,), jnp.int32),
            pltpu.VMEM((CH,), jnp.int32),
            pltpu.SemaphoreType.DMA,
        ],
        compiler_params=pltpu.CompilerParams(use_tc_tiling_on_sc=False),
    )


def kernel(x, edge_index, W_source, W_target):
    xpad = jnp.pad(x, ((0, NPAD - N), (0, 0)))
    srcx, tgtx = _mm_call(xpad, W_source, W_target)
    sidx = edge_index[0].reshape(-1)
    didx = edge_index[1].reshape(-1)
    partials = _sc_call()(srcx, sidx, didx)
    y = _combine_call(partials, tgtx)
    return y[:N, :D]


# speedup vs baseline: 8.0522x; 1.5875x over previous
import functools

import jax
import jax.numpy as jnp
from jax import lax
from jax.experimental import pallas as pl
from jax.experimental.pallas import tpu as pltpu
from jax.experimental.pallas import tpu_sc as plsc

N = 10000
E = 320000
D = 128
W = 144
NPAD = 10240
NC = 2
NS = 16
NW = NC * NS
EP_W = E // NW
CH = 104
NF = EP_W // CH
TAIL = EP_W - NF * CH
RPT = NPAD // NS
MM_BLK = 1024


def _mm_body(x_ref, ws_ref, wt_ref, src_ref, tgt_ref):
    xb = x_ref[...]
    s = jnp.dot(xb, ws_ref[...], preferred_element_type=jnp.float32)
    t = jnp.dot(xb, wt_ref[...], preferred_element_type=jnp.float32)
    lane = lax.broadcasted_iota(jnp.int32, (MM_BLK, W - D), 1)
    one_col = jnp.where(lane == 0, 1.0, 0.0).astype(jnp.float32)
    src_ref[...] = jnp.concatenate([s, one_col], axis=1)
    tgt_ref[...] = jnp.concatenate([t, jnp.zeros_like(one_col)], axis=1)


_mm_call = pl.pallas_call(
    _mm_body,
    grid=(NPAD // MM_BLK,),
    in_specs=[
        pl.BlockSpec((MM_BLK, D), lambda i: (i, 0)),
        pl.BlockSpec((D, D), lambda i: (0, 0)),
        pl.BlockSpec((D, D), lambda i: (0, 0)),
    ],
    out_specs=[
        pl.BlockSpec((MM_BLK, W), lambda i: (i, 0)),
        pl.BlockSpec((MM_BLK, W), lambda i: (i, 0)),
    ],
    out_shape=[
        jax.ShapeDtypeStruct((NPAD, W), jnp.float32),
        jax.ShapeDtypeStruct((NPAD, W), jnp.float32),
    ],
)


def _sc_body(
    srcx, sidx_h, didx_h, out, acc,
    rows0, rows1, si0, si1, di0, di1,
    rows_t, si_t, di_t,
    gs0, gs1, ss0, ss1, sem_t,
):
    c = lax.axis_index("c")
    s = lax.axis_index("s")
    wid = c * NS + s
    bufs = ((rows0, si0, di0, gs0, ss0), (rows1, si1, di1, gs1, ss1))

    z16 = jnp.zeros((16,), jnp.float32)

    def _zrow(i, carry):
        for j in range(W // 16):
            rows0[i, pl.ds(j * 16, 16)] = z16
        return carry

    ZCH = 80
    lax.fori_loop(0, ZCH, _zrow, 0)
    for k in range(RPT // ZCH):
        pltpu.sync_copy(
            rows0.at[pl.ds(0, ZCH)], acc.at[pl.ds(s * RPT + k * ZCH, ZCH)]
        )
    plsc.subcore_barrier()

    def _start(i, b):
        rows, si, di, gsem, _ = bufs[b]
        base = wid * EP_W + i * CH
        pltpu.sync_copy(sidx_h.at[pl.ds(base, CH)], si)
        pltpu.sync_copy(didx_h.at[pl.ds(base, CH)], di)
        pltpu.async_copy(srcx.at[si], rows, gsem)

    def _wait_gather(b):
        rows, si, _, gsem, _ = bufs[b]
        pltpu.make_async_copy(srcx.at[si], rows, gsem).wait()

    def _start_scatter(b):
        rows, _, di, _, ssem = bufs[b]
        pltpu.async_copy(rows, acc.at[di], ssem, add=True)

    def _wait_scatter(b):
        rows, _, di, _, ssem = bufs[b]
        pltpu.make_async_copy(rows, acc.at[di], ssem).wait()

    _start(0, 0)
    _start(1, 1)
    _wait_gather(0)
    _start_scatter(0)

    def _pair(p, carry):
        i0 = 2 * p + 2
        _wait_scatter(0)
        _start(i0, 0)
        _wait_gather(1)
        _start_scatter(1)
        _wait_scatter(1)
        _start(i0 + 1, 1)
        _wait_gather(0)
        _start_scatter(0)
        return carry

    lax.fori_loop(0, (NF - 2) // 2, _pair, 0)
    _wait_gather(1)
    _start_scatter(1)

    base = wid * EP_W + NF * CH
    pltpu.sync_copy(sidx_h.at[pl.ds(base, TAIL)], si_t)
    pltpu.sync_copy(didx_h.at[pl.ds(base, TAIL)], di_t)
    pltpu.async_copy(srcx.at[si_t], rows_t, sem_t).wait()
    pltpu.sync_copy(rows_t, acc.at[di_t], add=True)

    _wait_scatter(0)
    _wait_scatter(1)
    plsc.subcore_barrier()

    pltpu.sync_copy(acc.at[pl.ds(s * RPT, RPT)], out.at[c, pl.ds(s * RPT, RPT)])


def _combine_body(p_ref, t_ref, o_ref):
    p = p_ref[...]
    total = p[0] + p[1]
    lane = lax.broadcasted_iota(jnp.int32, total.shape, 1)
    cnt = jnp.sum(jnp.where(lane == D, total, 0.0), axis=1, keepdims=True)
    mean = total * (1.0 / jnp.maximum(cnt, 1.0))
    o_ref[...] = jnp.maximum(t_ref[...] + mean, 0.0)


_combine_call = pl.pallas_call(
    _combine_body,
    grid=(NPAD // MM_BLK,),
    in_specs=[
        pl.BlockSpec((NC, MM_BLK, W), lambda i: (0, i, 0)),
        pl.BlockSpec((MM_BLK, W), lambda i: (i, 0)),
    ],
    out_specs=pl.BlockSpec((MM_BLK, W), lambda i: (i, 0)),
    out_shape=jax.ShapeDtypeStruct((NPAD, W), jnp.float32),
)


@functools.cache
def _sc_call():
    mesh = plsc.VectorSubcoreMesh(
        core_axis_name="c", subcore_axis_name="s", num_cores=NC, num_subcores=NS
    )
    return pl.kernel(
        _sc_body,
        mesh=mesh,
        out_type=jax.ShapeDtypeStruct((NC, NPAD, W), jnp.float32),
        scratch_types=[
            pltpu.VMEM_SHARED((NPAD, W), jnp.float32),
            pltpu.VMEM((CH, W), jnp.float32),
            pltpu.VMEM((CH, W), jnp.float32),
            pltpu.VMEM((CH,), jnp.int32),
            pltpu.VMEM((CH,), jnp.int32),
            pltpu.VMEM((CH,), jnp.int32),
            pltpu.VMEM((CH,), jnp.int32),
            pltpu.VMEM((TAIL, W), jnp.float32),
            pltpu.VMEM((TAIL,), jnp.int32),
            pltpu.VMEM((TAIL,), jnp.int32),
            pltpu.SemaphoreType.DMA,
            pltpu.SemaphoreType.DMA,
            pltpu.SemaphoreType.DMA,
            pltpu.SemaphoreType.DMA,
            pltpu.SemaphoreType.DMA,
        ],
        compiler_params=pltpu.CompilerParams(use_tc_tiling_on_sc=False),
    )


def kernel(x, edge_index, W_source, W_target):
    xpad = jnp.pad(x, ((0, NPAD - N), (0, 0)))
    srcx, tgtx = _mm_call(xpad, W_source, W_target)
    sidx = edge_index[0].reshape(-1)
    didx = edge_index[1].reshape(-1)
    partials = _sc_call()(srcx, sidx, didx)
    y = _combine_call(partials, tgtx)
    return y[:N, :D]


# speedup vs baseline: 9.1357x; 1.1346x over previous
import functools

import jax
import jax.numpy as jnp
from jax import lax
from jax.experimental import pallas as pl
from jax.experimental.pallas import tpu as pltpu
from jax.experimental.pallas import tpu_sc as plsc

N = 10000
E = 320000
D = 128
W = 144
NPAD = 10240
NC = 2
NS = 16
NW = NC * NS
EP_W = E // NW
CH = 96
GC = 4
NF = EP_W // CH
NG = NF // GC
TAIL = EP_W - NF * CH
RPT = NPAD // NS
MM_BLK = 1024


def _mm_body(x_ref, ws_ref, src_ref):
    xb = x_ref[...]
    s = jnp.dot(xb, ws_ref[...], preferred_element_type=jnp.float32)
    lane = lax.broadcasted_iota(jnp.int32, (MM_BLK, W - D), 1)
    one_col = jnp.where(lane == 0, 1.0, 0.0).astype(jnp.float32)
    src_ref[...] = jnp.concatenate([s, one_col], axis=1)


_mm_call = pl.pallas_call(
    _mm_body,
    grid=(NPAD // MM_BLK,),
    in_specs=[
        pl.BlockSpec((MM_BLK, D), lambda i: (i, 0)),
        pl.BlockSpec((D, D), lambda i: (0, 0)),
    ],
    out_specs=pl.BlockSpec((MM_BLK, W), lambda i: (i, 0)),
    out_shape=jax.ShapeDtypeStruct((NPAD, W), jnp.float32),
)


def _sc_body(
    srcx, sidx4, didx4, stail, dtail, out, acc,
    rows0, rows1, sstg0, sstg1, dstg0, dstg1, si_t, di_t,
    sif0, sif1, dif0, dif1,
    gs0, gs1, ss0, ss1, is0, is1, sem_t,
):
    c = lax.axis_index("c")
    s = lax.axis_index("s")
    wid = c * NS + s
    rows = (rows0, rows1)
    gsem = (gs0, gs1)
    ssem = (ss0, ss1)
    sstg = (sstg0, sstg1)
    dstg = (dstg0, dstg1)
    isem = (is0, is1)
    sif = (sif0, sif1)
    dif = (dif0, dif1)

    z16 = jnp.zeros((16,), jnp.float32)

    def _zrow(i, carry):
        for j in range(W // 16):
            rows0[i, pl.ds(j * 16, 16)] = z16
        return carry

    ZCH = 80
    lax.fori_loop(0, ZCH, _zrow, 0)
    for k in range(RPT // ZCH):
        pltpu.sync_copy(
            rows0.at[pl.ds(0, ZCH)], acc.at[pl.ds(s * RPT + k * ZCH, ZCH)]
        )
    plsc.subcore_barrier()

    def _desc(m):
        m = m % (2 * GC)
        return (m // GC) % 2, m % GC, m % 2

    def _gstart(m):
        sb, j, rb = _desc(m)
        for v in range(CH // 16):
            sif[rb][pl.ds(v * 16, 16)] = sstg[sb][j, pl.ds(v * 16, 16)]
            dif[rb][pl.ds(v * 16, 16)] = dstg[sb][j, pl.ds(v * 16, 16)]
        pltpu.async_copy(srcx.at[sif[rb]], rows[rb], gsem[rb])

    def _gwait(m):
        _, _, rb = _desc(m)
        pltpu.make_async_copy(srcx.at[sif[rb]], rows[rb], gsem[rb]).wait()

    def _sstart(m):
        _, _, rb = _desc(m)
        pltpu.async_copy(rows[rb], acc.at[dif[rb]], ssem[rb], add=True)

    def _swait(m):
        _, _, rb = _desc(m)
        pltpu.make_async_copy(rows[rb], acc.at[dif[rb]], ssem[rb]).wait()

    def _stg_start(g, sb):
        pltpu.async_copy(sidx4.at[wid, g], sstg[sb], isem[sb])
        pltpu.async_copy(didx4.at[wid, g], dstg[sb], isem[sb])

    def _stg_wait(g, sb):
        pltpu.make_async_copy(sidx4.at[wid, g], sstg[sb], isem[sb]).wait()
        pltpu.make_async_copy(didx4.at[wid, g], dstg[sb], isem[sb]).wait()

    def _slot(p, m):
        _swait(m)
        if m == 1:
            _stg_start(2 * p + 1, 1)
        if m == 4:
            _stg_wait(2 * p + 1, 1)
        if m == 5:
            _stg_start(jnp.minimum(2 * p + 2, NG - 1), 0)
        _gstart(m)
        _gwait(m - 1)
        _sstart(m - 1)

    _stg_start(0, 0)
    _stg_wait(0, 0)
    _stg_start(1, 1)
    _gstart(0)
    _gstart(1)
    _gwait(0)
    _sstart(0)
    for m in range(2, 2 * GC):
        _slot(0, m)

    def _body(p, carry):
        _stg_wait(2 * p, 0)
        for m in range(2 * GC):
            _slot(p, m)
        return carry

    lax.fori_loop(1, NG // 2, _body, 0)

    _gwait(2 * GC - 1)
    _sstart(2 * GC - 1)
    _swait(2 * GC - 2)
    _swait(2 * GC - 1)
    _stg_wait(NG - 1, 0)

    pltpu.sync_copy(stail.at[wid], si_t)
    pltpu.sync_copy(dtail.at[wid], di_t)
    pltpu.async_copy(srcx.at[si_t], rows0.at[pl.ds(0, TAIL)], sem_t).wait()
    pltpu.sync_copy(rows0.at[pl.ds(0, TAIL)], acc.at[di_t], add=True)

    plsc.subcore_barrier()

    pltpu.sync_copy(acc.at[pl.ds(s * RPT, RPT)], out.at[c, pl.ds(s * RPT, RPT)])


@functools.cache
def _sc_call():
    mesh = plsc.VectorSubcoreMesh(
        core_axis_name="c", subcore_axis_name="s", num_cores=NC, num_subcores=NS
    )
    return pl.kernel(
        _sc_body,
        mesh=mesh,
        out_type=jax.ShapeDtypeStruct((NC, NPAD, W), jnp.float32),
        scratch_types=[
            pltpu.VMEM_SHARED((NPAD, W), jnp.float32),
            pltpu.VMEM((CH, W), jnp.float32),
            pltpu.VMEM((CH, W), jnp.float32),
            pltpu.VMEM((GC, CH), jnp.int32),
            pltpu.VMEM((GC, CH), jnp.int32),
            pltpu.VMEM((GC, CH), jnp.int32),
            pltpu.VMEM((GC, CH), jnp.int32),
            pltpu.VMEM((TAIL,), jnp.int32),
            pltpu.VMEM((TAIL,), jnp.int32),
            pltpu.VMEM((CH,), jnp.int32),
            pltpu.VMEM((CH,), jnp.int32),
            pltpu.VMEM((CH,), jnp.int32),
            pltpu.VMEM((CH,), jnp.int32),
            pltpu.SemaphoreType.DMA,
            pltpu.SemaphoreType.DMA,
            pltpu.SemaphoreType.DMA,
            pltpu.SemaphoreType.DMA,
            pltpu.SemaphoreType.DMA,
            pltpu.SemaphoreType.DMA,
            pltpu.SemaphoreType.DMA,
        ],
        compiler_params=pltpu.CompilerParams(use_tc_tiling_on_sc=False),
    )


def _combine_body(p_ref, x_ref, wt_ref, o_ref):
    p = p_ref[...]
    total = p[0] + p[1]
    lane = lax.broadcasted_iota(jnp.int32, total.shape, 1)
    cnt = jnp.sum(jnp.where(lane == D, total, 0.0), axis=1, keepdims=True)
    mean = total * (1.0 / jnp.maximum(cnt, 1.0))
    tgt = jnp.dot(x_ref[...], wt_ref[...], preferred_element_type=jnp.float32)
    o_ref[...] = jnp.maximum(tgt + mean[:, :D], 0.0)


_combine_call = pl.pallas_call(
    _combine_body,
    grid=(NPAD // MM_BLK,),
    in_specs=[
        pl.BlockSpec((NC, MM_BLK, W), lambda i: (0, i, 0)),
        pl.BlockSpec((MM_BLK, D), lambda i: (i, 0)),
        pl.BlockSpec((D, D), lambda i: (0, 0)),
    ],
    out_specs=pl.BlockSpec((MM_BLK, D), lambda i: (i, 0)),
    out_shape=jax.ShapeDtypeStruct((N, D), jnp.float32),
)


def kernel(x, edge_index, W_source, W_target):
    srcx = _mm_call(x, W_source)
    ei = edge_index.reshape(2, NW, EP_W)
    main = ei[:, :, : NF * CH].reshape(2, NW, NG, GC, CH)
    partials = _sc_call()(
        srcx, main[0], main[1], ei[0, :, NF * CH :], ei[1, :, NF * CH :]
    )
    return _combine_call(partials, x, W_target)


# speedup vs baseline: 9.7059x; 1.0624x over previous
import functools

import jax
import jax.numpy as jnp
from jax import lax
from jax.experimental import pallas as pl
from jax.experimental.pallas import tpu as pltpu
from jax.experimental.pallas import tpu_sc as plsc

N = 10000
E = 320000
D = 128
W = 144
NPAD = 10240
NC = 2
NS = 16
NW = NC * NS
EP_W = E // NW
CH = 96
GC = 4
NF = EP_W // CH
NG = NF // GC
TAIL = EP_W - NF * CH
RPT = NPAD // NS
MM_BLK = 1024


def _mm_body(x_ref, ws_ref, src_ref):
    xb = x_ref[...]
    s = jnp.dot(xb, ws_ref[...], preferred_element_type=jnp.float32)
    lane = lax.broadcasted_iota(jnp.int32, (MM_BLK, W - D), 1)
    one_col = jnp.where(lane == 0, 1.0, 0.0).astype(jnp.float32)
    src_ref[...] = jnp.concatenate([s, one_col], axis=1)


_mm_call = pl.pallas_call(
    _mm_body,
    grid=(NPAD // MM_BLK,),
    in_specs=[
        pl.BlockSpec((MM_BLK, D), lambda i: (i, 0)),
        pl.BlockSpec((D, D), lambda i: (0, 0)),
    ],
    out_specs=pl.BlockSpec((MM_BLK, W), lambda i: (i, 0)),
    out_shape=jax.ShapeDtypeStruct((NPAD, W), jnp.float32),
)


def _sc_body(
    srcx, eidx, out, acc,
    rows0, rows1, sstg0, sstg1, dstg0, dstg1, si_t, di_t,
    sif0, sif1, dif0, dif1,
    gs0, gs1, ss0, ss1, is0, is1, sem_t,
):
    c = lax.axis_index("c")
    s = lax.axis_index("s")
    wid = c * NS + s
    rows = (rows0, rows1)
    gsem = (gs0, gs1)
    ssem = (ss0, ss1)
    sstg = (sstg0, sstg1)
    dstg = (dstg0, dstg1)
    isem = (is0, is1)
    sif = (sif0, sif1)
    dif = (dif0, dif1)

    z16 = jnp.zeros((16,), jnp.float32)

    def _zrow(i, carry):
        for j in range(W // 16):
            rows0[i, pl.ds(j * 16, 16)] = z16
        return carry

    ZCH = 80
    lax.fori_loop(0, ZCH, _zrow, 0)
    for k in range(RPT // ZCH):
        pltpu.sync_copy(
            rows0.at[pl.ds(0, ZCH)], acc.at[pl.ds(s * RPT + k * ZCH, ZCH)]
        )
    plsc.subcore_barrier()

    def _desc(m):
        m = m % (2 * GC)
        return (m // GC) % 2, m % GC, m % 2

    def _gstart(m):
        sb, j, rb = _desc(m)
        for v in range(CH // 16):
            sif[rb][pl.ds(v * 16, 16)] = sstg[sb][pl.ds(j * CH + v * 16, 16)]
            dif[rb][pl.ds(v * 16, 16)] = dstg[sb][pl.ds(j * CH + v * 16, 16)]
        pltpu.async_copy(srcx.at[sif[rb]], rows[rb], gsem[rb])

    def _gwait(m):
        _, _, rb = _desc(m)
        pltpu.make_async_copy(srcx.at[sif[rb]], rows[rb], gsem[rb]).wait()

    def _sstart(m):
        _, _, rb = _desc(m)
        pltpu.async_copy(rows[rb], acc.at[dif[rb]], ssem[rb], add=True)

    def _swait(m):
        _, _, rb = _desc(m)
        pltpu.make_async_copy(rows[rb], acc.at[dif[rb]], ssem[rb]).wait()

    GCH = GC * CH

    def _stg_start(g, sb):
        base = wid * EP_W + g * GCH
        pltpu.async_copy(eidx.at[0, pl.ds(base, GCH)], sstg[sb], isem[sb])
        pltpu.async_copy(eidx.at[1, pl.ds(base, GCH)], dstg[sb], isem[sb])

    def _stg_wait(g, sb):
        base = wid * EP_W + g * GCH
        pltpu.make_async_copy(eidx.at[0, pl.ds(base, GCH)], sstg[sb], isem[sb]).wait()
        pltpu.make_async_copy(eidx.at[1, pl.ds(base, GCH)], dstg[sb], isem[sb]).wait()

    def _slot(p, m):
        _swait(m)
        if m == 1:
            _stg_start(2 * p + 1, 1)
        if m == 4:
            _stg_wait(2 * p + 1, 1)
        if m == 5:
            _stg_start(jnp.minimum(2 * p + 2, NG - 1), 0)
        _gstart(m)
        _gwait(m - 1)
        _sstart(m - 1)

    _stg_start(0, 0)
    _stg_wait(0, 0)
    _stg_start(1, 1)
    _gstart(0)
    _gstart(1)
    _gwait(0)
    _sstart(0)
    for m in range(2, 2 * GC):
        _slot(0, m)

    def _body(p, carry):
        _stg_wait(2 * p, 0)
        for m in range(2 * GC):
            _slot(p, m)
        return carry

    lax.fori_loop(1, NG // 2, _body, 0)

    _gwait(2 * GC - 1)
    _sstart(2 * GC - 1)
    _swait(2 * GC - 2)
    _swait(2 * GC - 1)
    _stg_wait(NG - 1, 0)

    tbase = wid * EP_W + NF * CH
    pltpu.sync_copy(eidx.at[0, pl.ds(tbase, TAIL)], si_t)
    pltpu.sync_copy(eidx.at[1, pl.ds(tbase, TAIL)], di_t)
    pltpu.async_copy(srcx.at[si_t], rows0.at[pl.ds(0, TAIL)], sem_t).wait()
    pltpu.sync_copy(rows0.at[pl.ds(0, TAIL)], acc.at[di_t], add=True)

    plsc.subcore_barrier()

    pltpu.sync_copy(acc.at[pl.ds(s * RPT, RPT)], out.at[c, pl.ds(s * RPT, RPT)])


@functools.cache
def _sc_call():
    mesh = plsc.VectorSubcoreMesh(
        core_axis_name="c", subcore_axis_name="s", num_cores=NC, num_subcores=NS
    )
    return pl.kernel(
        _sc_body,
        mesh=mesh,
        out_type=jax.ShapeDtypeStruct((NC, NPAD, W), jnp.float32),
        scratch_types=[
            pltpu.VMEM_SHARED((NPAD, W), jnp.float32),
            pltpu.VMEM((CH, W), jnp.float32),
            pltpu.VMEM((CH, W), jnp.float32),
            pltpu.VMEM((GC * CH,), jnp.int32),
            pltpu.VMEM((GC * CH,), jnp.int32),
            pltpu.VMEM((GC * CH,), jnp.int32),
            pltpu.VMEM((GC * CH,), jnp.int32),
            pltpu.VMEM((TAIL,), jnp.int32),
            pltpu.VMEM((TAIL,), jnp.int32),
            pltpu.VMEM((CH,), jnp.int32),
            pltpu.VMEM((CH,), jnp.int32),
            pltpu.VMEM((CH,), jnp.int32),
            pltpu.VMEM((CH,), jnp.int32),
            pltpu.SemaphoreType.DMA,
            pltpu.SemaphoreType.DMA,
            pltpu.SemaphoreType.DMA,
            pltpu.SemaphoreType.DMA,
            pltpu.SemaphoreType.DMA,
            pltpu.SemaphoreType.DMA,
            pltpu.SemaphoreType.DMA,
        ],
        compiler_params=pltpu.CompilerParams(use_tc_tiling_on_sc=False),
    )


def _combine_body(p_ref, x_ref, wt_ref, o_ref):
    p = p_ref[...]
    total = p[0] + p[1]
    lane = lax.broadcasted_iota(jnp.int32, total.shape, 1)
    cnt = jnp.sum(jnp.where(lane == D, total, 0.0), axis=1, keepdims=True)
    mean = total * (1.0 / jnp.maximum(cnt, 1.0))
    tgt = jnp.dot(x_ref[...], wt_ref[...], preferred_element_type=jnp.float32)
    o_ref[...] = jnp.maximum(tgt + mean[:, :D], 0.0)


_combine_call = pl.pallas_call(
    _combine_body,
    grid=(NPAD // MM_BLK,),
    in_specs=[
        pl.BlockSpec((NC, MM_BLK, W), lambda i: (0, i, 0)),
        pl.BlockSpec((MM_BLK, D), lambda i: (i, 0)),
        pl.BlockSpec((D, D), lambda i: (0, 0)),
    ],
    out_specs=pl.BlockSpec((MM_BLK, D), lambda i: (i, 0)),
    out_shape=jax.ShapeDtypeStruct((N, D), jnp.float32),
)


def kernel(x, edge_index, W_source, W_target):
    srcx = _mm_call(x, W_source)
    partials = _sc_call()(srcx, edge_index)
    return _combine_call(partials, x, W_target)


# speedup vs baseline: 12.3182x; 1.2691x over previous
import functools

import jax
import jax.numpy as jnp
from jax import lax
from jax.experimental import pallas as pl
from jax.experimental.pallas import tpu as pltpu
from jax.experimental.pallas import tpu_sc as plsc

N = 10000
E = 320000
D = 128
NPAD = 10240
NC = 2
NS = 16
NW = NC * NS
CH = 128
ER = E // CH
NF = ER // NW
NTAIL = ER - NF * NW
CW = 16
RPT = NPAD // NS
MM_BLK = 1024


def _mm_body(x_ref, ws_ref, src_ref):
    src_ref[...] = jnp.dot(
        x_ref[...], ws_ref[...], preferred_element_type=jnp.float32
    )


_mm_call = pl.pallas_call(
    _mm_body,
    grid=(NPAD // MM_BLK,),
    in_specs=[
        pl.BlockSpec((MM_BLK, D), lambda i: (i, 0)),
        pl.BlockSpec((D, D), lambda i: (0, 0)),
    ],
    out_specs=pl.BlockSpec((MM_BLK, D), lambda i: (i, 0)),
    out_shape=jax.ShapeDtypeStruct((NPAD, D), jnp.float32),
)


def _sc_body(
    srcx, se2, de2, out, outc, acc, accc,
    rows0, rows1, si0, si1, si2, si3, di0, di1, di2, di3, ones, zc,
    gs0, gs1, ss0, ss1, is0, is1, is2, is3, sem_t,
):
    c = lax.axis_index("c")
    s = lax.axis_index("s")
    wid = c * NS + s
    rows = (rows0, rows1)
    gsem = (gs0, gs1)
    ssem = (ss0, ss1)
    si = (si0, si1, si2, si3)
    di = (di0, di1, di2, di3)
    isem = (is0, is1, is2, is3)

    z16 = jnp.zeros((16,), jnp.float32)
    lane = lax.iota(jnp.int32, 16)
    one16 = jnp.where(lane == 0, 1.0, 0.0).astype(jnp.float32)

    def _zrow(i, carry):
        for j in range(D // 16):
            rows0[i, pl.ds(j * 16, 16)] = z16
        ones[i, pl.ds(0, 16)] = one16
        return carry

    ZCH = 80
    lax.fori_loop(0, ZCH, _zrow, 0)

    def _zcrow(i, carry):
        zc[i, pl.ds(0, 16)] = z16
        return carry

    lax.fori_loop(0, 16, _zcrow, 0)

    def _onerow(i, carry):
        ones[i, pl.ds(0, 16)] = one16
        return carry

    lax.fori_loop(ZCH, CH, _onerow, 0)
    for k in range(RPT // ZCH):
        pltpu.sync_copy(
            rows0.at[pl.ds(0, ZCH)], acc.at[pl.ds(s * RPT + k * ZCH, ZCH)]
        )
    for k in range(RPT // 16):
        pltpu.sync_copy(zc, accc.at[pl.ds(s * RPT + k * 16, 16)])
    plsc.subcore_barrier()

    def _istart(k, m):
        pltpu.async_copy(se2.at[k * NW + wid], si[m], isem[m])
        pltpu.async_copy(de2.at[k * NW + wid], di[m], isem[m])

    def _iwait(k, m):
        pltpu.make_async_copy(se2.at[k * NW + wid], si[m], isem[m]).wait()
        pltpu.make_async_copy(de2.at[k * NW + wid], di[m], isem[m]).wait()

    def _gstart(m, rb):
        pltpu.async_copy(srcx.at[si[m]], rows[rb], gsem[rb])

    def _gwait(m, rb):
        pltpu.make_async_copy(srcx.at[si[m]], rows[rb], gsem[rb]).wait()

    def _sstart(m, rb):
        pltpu.async_copy(rows[rb], acc.at[di[m]], ssem[rb], add=True)
        pltpu.async_copy(ones, accc.at[di[m]], ssem[rb], add=True)

    def _swait(m, rb):
        pltpu.make_async_copy(rows[rb], acc.at[di[m]], ssem[rb]).wait()
        pltpu.make_async_copy(ones, accc.at[di[m]], ssem[rb]).wait()

    _istart(0, 0)
    _istart(1, 1)
    _istart(2, 2)
    _istart(3, 3)
    _iwait(0, 0)
    _gstart(0, 0)
    _iwait(1, 1)
    _gstart(1, 1)
    _gwait(0, 0)
    _sstart(0, 0)

    def _body(p, carry):
        for t in range(4):
            k = 2 + 4 * p + t
            m, rb = (2 + t) % 4, t % 2
            _swait((m + 2) % 4, rb)
            _istart(jnp.minimum(k + 2, NF - 1), (m + 2) % 4)
            _iwait(k, m)
            _gstart(m, rb)
            _gwait((m + 3) % 4, (rb + 1) % 2)
            _sstart((m + 3) % 4, (rb + 1) % 2)
        return carry

    lax.fori_loop(0, (NF - 2) // 4, _body, 0)

    _gwait(1, 1)
    _sstart(1, 1)
    _swait(0, 0)
    _swait(1, 1)
    _iwait(NF - 1, 2)
    _iwait(NF - 1, 3)

    @pl.when(wid < NTAIL)
    def _tail():
        row = NF * NW + wid
        pltpu.async_copy(se2.at[row], si0, sem_t)
        pltpu.async_copy(de2.at[row], di0, sem_t)
        pltpu.make_async_copy(se2.at[row], si0, sem_t).wait()
        pltpu.make_async_copy(de2.at[row], di0, sem_t).wait()
        pltpu.async_copy(srcx.at[si0], rows0, sem_t)
        pltpu.make_async_copy(srcx.at[si0], rows0, sem_t).wait()
        pltpu.sync_copy(rows0, acc.at[di0], add=True)
        pltpu.sync_copy(ones, accc.at[di0], add=True)

    plsc.subcore_barrier()

    pltpu.sync_copy(acc.at[pl.ds(s * RPT, RPT)], out.at[c, pl.ds(s * RPT, RPT)])
    pltpu.sync_copy(
        accc.at[pl.ds(s * RPT, RPT)], outc.at[c, pl.ds(s * RPT, RPT)]
    )


@functools.cache
def _sc_call():
    mesh = plsc.VectorSubcoreMesh(
        core_axis_name="c", subcore_axis_name="s", num_cores=NC, num_subcores=NS
    )
    return pl.kernel(
        _sc_body,
        mesh=mesh,
        out_type=(
            jax.ShapeDtypeStruct((NC, NPAD, D), jnp.float32),
            jax.ShapeDtypeStruct((NC, NPAD, CW), jnp.float32),
        ),
        scratch_types=[
            pltpu.VMEM_SHARED((NPAD, D), jnp.float32),
            pltpu.VMEM_SHARED((NPAD, CW), jnp.float32),
            pltpu.VMEM((CH, D), jnp.float32),
            pltpu.VMEM((CH, D), jnp.float32),
            pltpu.VMEM((CH,), jnp.int32),
            pltpu.VMEM((CH,), jnp.int32),
            pltpu.VMEM((CH,), jnp.int32),
            pltpu.VMEM((CH,), jnp.int32),
            pltpu.VMEM((CH,), jnp.int32),
            pltpu.VMEM((CH,), jnp.int32),
            pltpu.VMEM((CH,), jnp.int32),
            pltpu.VMEM((CH,), jnp.int32),
            pltpu.VMEM((CH, CW), jnp.float32),
            pltpu.VMEM((16, CW), jnp.float32),
            pltpu.SemaphoreType.DMA,
            pltpu.SemaphoreType.DMA,
            pltpu.SemaphoreType.DMA,
            pltpu.SemaphoreType.DMA,
            pltpu.SemaphoreType.DMA,
            pltpu.SemaphoreType.DMA,
            pltpu.SemaphoreType.DMA,
            pltpu.SemaphoreType.DMA,
            pltpu.SemaphoreType.DMA,
        ],
        compiler_params=pltpu.CompilerParams(use_tc_tiling_on_sc=False),
    )


def _combine_body(p_ref, c_ref, x_ref, wt_ref, o_ref):
    p = p_ref[...]
    cc = c_ref[...]
    total = p[0] + p[1]
    cnt = jnp.sum(cc[0] + cc[1], axis=1, keepdims=True)
    mean = total * (1.0 / jnp.maximum(cnt, 1.0))
    tgt = jnp.dot(x_ref[...], wt_ref[...], preferred_element_type=jnp.float32)
    o_ref[...] = jnp.maximum(tgt + mean, 0.0)


_combine_call = pl.pallas_call(
    _combine_body,
    grid=(NPAD // MM_BLK,),
    in_specs=[
        pl.BlockSpec((NC, MM_BLK, D), lambda i: (0, i, 0)),
        pl.BlockSpec((NC, MM_BLK, CW), lambda i: (0, i, 0)),
        pl.BlockSpec((MM_BLK, D), lambda i: (i, 0)),
        pl.BlockSpec((D, D), lambda i: (0, 0)),
    ],
    out_specs=pl.BlockSpec((MM_BLK, D), lambda i: (i, 0)),
    out_shape=jax.ShapeDtypeStruct((N, D), jnp.float32),
)


def kernel(x, edge_index, W_source, W_target):
    srcx = _mm_call(x, W_source)
    se2 = edge_index[0].reshape(ER, CH)
    de2 = edge_index[1].reshape(ER, CH)
    partials, counts = _sc_call()(srcx, se2, de2)
    return _combine_call(partials, counts, x, W_target)


# speedup vs baseline: 13.0058x; 1.0558x over previous
import functools

import jax
import jax.numpy as jnp
from jax import lax
from jax.experimental import pallas as pl
from jax.experimental.pallas import tpu as pltpu
from jax.experimental.pallas import tpu_sc as plsc

N = 10000
E = 320000
D = 128
NPAD = 10240
NC = 2
NS = 16
NW = NC * NS
CH = 128
ER = E // CH
NF = ER // NW
NTAIL = ER - NF * NW
CW = 16
RPT = NPAD // NS
MM_BLK = 1024


def _mm_body(x_ref, ws_ref, src_ref):
    src_ref[...] = jnp.dot(
        x_ref[...], ws_ref[...], preferred_element_type=jnp.float32
    )


_mm_call = pl.pallas_call(
    _mm_body,
    grid=(NPAD // MM_BLK,),
    in_specs=[
        pl.BlockSpec((MM_BLK, D), lambda i: (i, 0)),
        pl.BlockSpec((D, D), lambda i: (0, 0)),
    ],
    out_specs=pl.BlockSpec((MM_BLK, D), lambda i: (i, 0)),
    out_shape=jax.ShapeDtypeStruct((NPAD, D), jnp.float32),
)


def _sc_body(
    srcx, se2, out, outc, acc, accc,
    rows0, rows1, si0, si1, si2, si3, di0, di1, di2, di3, ones, zc,
    gs0, gs1, ss0, ss1, is0, is1, is2, is3, sem_t,
):
    c = lax.axis_index("c")
    s = lax.axis_index("s")
    wid = c * NS + s
    rows = (rows0, rows1)
    gsem = (gs0, gs1)
    ssem = (ss0, ss1)
    si = (si0, si1, si2, si3)
    di = (di0, di1, di2, di3)
    isem = (is0, is1, is2, is3)

    z16 = jnp.zeros((16,), jnp.float32)
    lane = lax.iota(jnp.int32, 16)
    one16 = jnp.where(lane == 0, 1.0, 0.0).astype(jnp.float32)

    def _zrow(i, carry):
        for j in range(D // 16):
            rows0[i, pl.ds(j * 16, 16)] = z16
        ones[i, pl.ds(0, 16)] = one16
        return carry

    ZCH = 80
    lax.fori_loop(0, ZCH, _zrow, 0)

    def _zcrow(i, carry):
        zc[i, pl.ds(0, 16)] = z16
        return carry

    lax.fori_loop(0, 16, _zcrow, 0)

    def _onerow(i, carry):
        ones[i, pl.ds(0, 16)] = one16
        return carry

    lax.fori_loop(ZCH, CH, _onerow, 0)
    for k in range(RPT // ZCH):
        pltpu.sync_copy(
            rows0.at[pl.ds(0, ZCH)], acc.at[pl.ds(s * RPT + k * ZCH, ZCH)]
        )
    for k in range(RPT // 16):
        pltpu.sync_copy(zc, accc.at[pl.ds(s * RPT + k * 16, 16)])
    plsc.subcore_barrier()

    def _istart(k, m):
        pltpu.async_copy(se2.at[k * NW + wid], si[m], isem[m])
        pltpu.async_copy(se2.at[ER + k * NW + wid], di[m], isem[m])

    def _iwait(k, m):
        pltpu.make_async_copy(se2.at[k * NW + wid], si[m], isem[m]).wait()
        pltpu.make_async_copy(se2.at[ER + k * NW + wid], di[m], isem[m]).wait()

    def _gstart(m, rb):
        pltpu.async_copy(srcx.at[si[m]], rows[rb], gsem[rb])

    def _gwait(m, rb):
        pltpu.make_async_copy(srcx.at[si[m]], rows[rb], gsem[rb]).wait()

    def _sstart(m, rb):
        pltpu.async_copy(rows[rb], acc.at[di[m]], ssem[rb], add=True)
        pltpu.async_copy(ones, accc.at[di[m]], ssem[rb], add=True)

    def _swait(m, rb):
        pltpu.make_async_copy(rows[rb], acc.at[di[m]], ssem[rb]).wait()
        pltpu.make_async_copy(ones, accc.at[di[m]], ssem[rb]).wait()

    _istart(0, 0)
    _istart(1, 1)
    _istart(2, 2)
    _istart(3, 3)
    _iwait(0, 0)
    _gstart(0, 0)
    _iwait(1, 1)
    _gstart(1, 1)
    _gwait(0, 0)
    _sstart(0, 0)

    def _body(p, carry):
        for t in range(4):
            k = 2 + 4 * p + t
            m, rb = (2 + t) % 4, t % 2
            _swait((m + 2) % 4, rb)
            _istart(jnp.minimum(k + 2, NF - 1), (m + 2) % 4)
            _iwait(k, m)
            _gstart(m, rb)
            _gwait((m + 3) % 4, (rb + 1) % 2)
            _sstart((m + 3) % 4, (rb + 1) % 2)
        return carry

    lax.fori_loop(0, (NF - 2) // 4, _body, 0)

    _gwait(1, 1)
    _sstart(1, 1)
    _swait(0, 0)
    _swait(1, 1)
    _iwait(NF - 1, 2)
    _iwait(NF - 1, 3)

    @pl.when(wid < NTAIL)
    def _tail():
        row = NF * NW + wid
        pltpu.async_copy(se2.at[row], si0, sem_t)
        pltpu.async_copy(se2.at[ER + row], di0, sem_t)
        pltpu.make_async_copy(se2.at[row], si0, sem_t).wait()
        pltpu.make_async_copy(se2.at[ER + row], di0, sem_t).wait()
        pltpu.async_copy(srcx.at[si0], rows0, sem_t)
        pltpu.make_async_copy(srcx.at[si0], rows0, sem_t).wait()
        pltpu.sync_copy(rows0, acc.at[di0], add=True)
        pltpu.sync_copy(ones, accc.at[di0], add=True)

    plsc.subcore_barrier()

    pltpu.sync_copy(acc.at[pl.ds(s * RPT, RPT)], out.at[c, pl.ds(s * RPT, RPT)])
    pltpu.sync_copy(
        accc.at[pl.ds(s * RPT, RPT)], outc.at[c, pl.ds(s * RPT, RPT)]
    )


@functools.cache
def _sc_call():
    mesh = plsc.VectorSubcoreMesh(
        core_axis_name="c", subcore_axis_name="s", num_cores=NC, num_subcores=NS
    )
    return pl.kernel(
        _sc_body,
        mesh=mesh,
        out_type=(
            jax.ShapeDtypeStruct((NC, NPAD, D), jnp.float32),
            jax.ShapeDtypeStruct((NC, NPAD, CW), jnp.float32),
        ),
        scratch_types=[
            pltpu.VMEM_SHARED((NPAD, D), jnp.float32),
            pltpu.VMEM_SHARED((NPAD, CW), jnp.float32),
            pltpu.VMEM((CH, D), jnp.float32),
            pltpu.VMEM((CH, D), jnp.float32),
            pltpu.VMEM((CH,), jnp.int32),
            pltpu.VMEM((CH,), jnp.int32),
            pltpu.VMEM((CH,), jnp.int32),
            pltpu.VMEM((CH,), jnp.int32),
            pltpu.VMEM((CH,), jnp.int32),
            pltpu.VMEM((CH,), jnp.int32),
            pltpu.VMEM((CH,), jnp.int32),
            pltpu.VMEM((CH,), jnp.int32),
            pltpu.VMEM((CH, CW), jnp.float32),
            pltpu.VMEM((16, CW), jnp.float32),
            pltpu.SemaphoreType.DMA,
            pltpu.SemaphoreType.DMA,
            pltpu.SemaphoreType.DMA,
            pltpu.SemaphoreType.DMA,
            pltpu.SemaphoreType.DMA,
            pltpu.SemaphoreType.DMA,
            pltpu.SemaphoreType.DMA,
            pltpu.SemaphoreType.DMA,
            pltpu.SemaphoreType.DMA,
        ],
        compiler_params=pltpu.CompilerParams(use_tc_tiling_on_sc=False),
    )


def _combine_body(p_ref, c_ref, x_ref, wt_ref, o_ref):
    p = p_ref[...]
    cc = c_ref[...]
    total = p[0] + p[1]
    cnt = jnp.sum(cc[0] + cc[1], axis=1, keepdims=True)
    mean = total * (1.0 / jnp.maximum(cnt, 1.0))
    tgt = jnp.dot(x_ref[...], wt_ref[...], preferred_element_type=jnp.float32)
    o_ref[...] = jnp.maximum(tgt + mean, 0.0)


_combine_call = pl.pallas_call(
    _combine_body,
    grid=(NPAD // MM_BLK,),
    in_specs=[
        pl.BlockSpec((NC, MM_BLK, D), lambda i: (0, i, 0)),
        pl.BlockSpec((NC, MM_BLK, CW), lambda i: (0, i, 0)),
        pl.BlockSpec((MM_BLK, D), lambda i: (i, 0)),
        pl.BlockSpec((D, D), lambda i: (0, 0)),
    ],
    out_specs=pl.BlockSpec((MM_BLK, D), lambda i: (i, 0)),
    out_shape=jax.ShapeDtypeStruct((N, D), jnp.float32),
)


def kernel(x, edge_index, W_source, W_target):
    srcx = _mm_call(x, W_source)
    e2 = edge_index.reshape(2 * ER, CH)
    partials, counts = _sc_call()(srcx, e2)
    return _combine_call(partials, counts, x, W_target)
